# baseline (speedup 1.0000x reference)
import functools
import math

import jax
import jax.numpy as jnp
from jax import lax
from jax.experimental import pallas as pl
from jax.experimental.pallas import tpu as pltpu
from jax.experimental.pallas import tpu_sc as plsc

D_MODEL = 64
SCALE = math.sqrt(D_MODEL)

_info = plsc.get_sparse_core_info()
_NC, _NS, _L = _info.num_cores, _info.num_subcores, _info.num_lanes
_NW = _NC * _NS

_CHUNK = 512


def _make_gather(B: int):
  assert B % (_NW * _CHUNK) == 0
  rows_per_w = B // _NW
  n_chunks = rows_per_w // _CHUNK
  mesh = plsc.VectorSubcoreMesh(core_axis_name="c", subcore_axis_name="s")

  @functools.partial(
      pl.kernel,
      mesh=mesh,
      compiler_params=pltpu.CompilerParams(use_tc_tiling_on_sc=False),
      out_type=jax.ShapeDtypeStruct((B, D_MODEL), jnp.float32),
      scratch_types=[
          pltpu.VMEM((_CHUNK,), jnp.int32),
          pltpu.VMEM((_CHUNK, D_MODEL), jnp.float32),
          pltpu.SemaphoreType.DMA,
      ],
  )
  def gather_kernel(table_hbm, idx_hbm, out_hbm, idx_v, rows_v, sem):
    wid = lax.axis_index("s") * _NC + lax.axis_index("c")
    base = wid * rows_per_w

    def chunk_body(g, carry):
      start = base + g * _CHUNK
      pltpu.sync_copy(idx_hbm.at[pl.ds(start, _CHUNK)], idx_v)
      pltpu.async_copy(table_hbm.at[idx_v], rows_v, sem).wait()

      def row_body(i, c):
        for j in range(D_MODEL // _L):
          sl = pl.ds(j * _L, _L)
          rows_v[i, sl] = rows_v[i, sl] * SCALE
        return c

      lax.fori_loop(0, _CHUNK, row_body, 0, unroll=4)
      pltpu.sync_copy(rows_v, out_hbm.at[pl.ds(start, _CHUNK)])
      return carry

    lax.fori_loop(0, n_chunks, chunk_body, 0)

  return gather_kernel


def kernel(x, table):
  B = x.size
  idx = x.reshape(-1).astype(jnp.int32)
  out = _make_gather(B)(table, idx)
  return out.reshape(*x.shape, D_MODEL)


# speedup vs baseline: 1.0919x; 1.0919x over previous
import functools
import math

import jax
import jax.numpy as jnp
from jax import lax
from jax.experimental import pallas as pl
from jax.experimental.pallas import tpu as pltpu
from jax.experimental.pallas import tpu_sc as plsc

D_MODEL = 64
SCALE = math.sqrt(D_MODEL)

_info = plsc.get_sparse_core_info()
_NC, _NS, _L = _info.num_cores, _info.num_subcores, _info.num_lanes
_NW = _NC * _NS

_CHUNK = 320


def _make_gather(B: int):
  assert B % (_NW * 2 * _CHUNK) == 0
  rows_per_w = B // _NW
  n_chunks = rows_per_w // _CHUNK
  n_pairs = n_chunks // 2
  mesh = plsc.VectorSubcoreMesh(core_axis_name="c", subcore_axis_name="s")

  @functools.partial(
      pl.kernel,
      mesh=mesh,
      compiler_params=pltpu.CompilerParams(use_tc_tiling_on_sc=False),
      out_type=jax.ShapeDtypeStruct((B, D_MODEL), jnp.float32),
      scratch_types=[
          pltpu.VMEM((rows_per_w,), jnp.int32),
          pltpu.VMEM((_CHUNK, D_MODEL), jnp.float32),
          pltpu.VMEM((_CHUNK, D_MODEL), jnp.float32),
          pltpu.VMEM((_CHUNK, D_MODEL), jnp.float32),
          pltpu.VMEM((_CHUNK, D_MODEL), jnp.float32),
          pltpu.SemaphoreType.DMA,
          pltpu.SemaphoreType.DMA,
          pltpu.SemaphoreType.DMA,
          pltpu.SemaphoreType.DMA,
      ],
  )
  def gather_kernel(table_hbm, idx_hbm, out_hbm, idx_v, g0, g1, s0, s1,
                    sem_g0, sem_g1, sem_s0, sem_s1):
    wid = lax.axis_index("s") * _NC + lax.axis_index("c")
    base = wid * rows_per_w
    gbuf = (g0, g1)
    sbuf = (s0, s1)
    sem_g = (sem_g0, sem_g1)
    sem_s = (sem_s0, sem_s1)

    def idx_slice(c):
      return idx_v.at[pl.ds(c * _CHUNK, _CHUNK)]

    def fire_gather(c, b):
      pltpu.async_copy(table_hbm.at[idx_slice(c)], gbuf[b], sem_g[b])

    def wait_gather(c, b):
      pltpu.make_async_copy(table_hbm.at[idx_slice(c)], gbuf[b],
                            sem_g[b]).wait()

    def fire_store(c, b):
      pltpu.async_copy(sbuf[b], out_hbm.at[pl.ds(base + c * _CHUNK, _CHUNK)],
                       sem_s[b])

    def wait_store(c, b):
      pltpu.make_async_copy(sbuf[b],
                            out_hbm.at[pl.ds(base + c * _CHUNK, _CHUNK)],
                            sem_s[b]).wait()

    def scale_chunk(b):
      g_ref, s_ref = gbuf[b], sbuf[b]

      @plsc.parallel_loop(0, _CHUNK, unroll=8)
      def _(i):
        for j in range(D_MODEL // _L):
          sl = pl.ds(j * _L, _L)
          s_ref[i, sl] = g_ref[i, sl] * SCALE

    pltpu.sync_copy(idx_hbm.at[pl.ds(base, rows_per_w)], idx_v)
    fire_gather(0, 0)
    fire_gather(1, 1)

    def pair_body(p, carry):
      for b in range(2):
        c = 2 * p + b
        wait_gather(c, b)

        @pl.when(p >= 1)
        def _():
          wait_store(c - 2, b)

        scale_chunk(b)
        fire_store(c, b)

        @pl.when(p <= n_pairs - 2)
        def _():
          fire_gather(c + 2, b)

      return carry

    lax.fori_loop(0, n_pairs, pair_body, 0)
    wait_store(n_chunks - 2, 0)
    wait_store(n_chunks - 1, 1)

  return gather_kernel


def kernel(x, table):
  B = x.size
  idx = x.reshape(-1).astype(jnp.int32)
  out = _make_gather(B)(table, idx)
  return out.reshape(*x.shape, D_MODEL)
